# baseline (speedup 1.0000x reference)
import math
import functools

import jax
import jax.numpy as jnp
import numpy as np
from jax import lax
from jax.experimental import pallas as pl
from jax.experimental.pallas import tpu as pltpu
from jax.experimental.pallas import tpu_sc as plsc

BSZ = 4
SEQ = 8192
NUM_TOKENS = BSZ * SEQ
NW = 32
CHUNK = NUM_TOKENS // NW
CPR = SEQ // CHUNK
LANES = 16

EMBEDDING_DIM = 1024
HALF_DIM = EMBEDDING_DIM // 2
SEQ_BLOCK = 512

_EMB_SCALE = math.log(10000.0) / (HALF_DIM - 1)
_FREQS = np.exp(np.arange(HALF_DIM, dtype=np.float32) * -_EMB_SCALE).astype(np.float32)


def _sc_pos_body(mask_hbm, pos_hbm, row_v, pos_v, sem):
    wid = lax.axis_index("s") * 2 + lax.axis_index("c")
    b = wid // CPR
    c = wid % CPR
    row_base = b * SEQ
    cbase = c * CHUNK

    pltpu.sync_copy(mask_hbm.at[pl.ds(row_base, SEQ)], row_v)

    def _ofs(i, acc):
        return acc + jnp.sum(row_v[pl.ds(i * LANES, LANES)])

    carry = lax.fori_loop(0, c * (CHUNK // LANES), _ofs, jnp.int32(0))

    for i in range(CHUNK // LANES):
        v = row_v[pl.ds(cbase + i * LANES, LANES)]
        cum = plsc.cumsum(v) + carry
        carry = carry + jnp.sum(v)
        pos_v[pl.ds(i * LANES, LANES)] = (cum * v).astype(jnp.float32)

    pltpu.async_copy(pos_v, pos_hbm.at[pl.ds(row_base + cbase, CHUNK)], sem).wait()


_sc_positions = functools.partial(
    pl.kernel,
    out_type=jax.ShapeDtypeStruct((NUM_TOKENS,), jnp.float32),
    mesh=plsc.VectorSubcoreMesh(core_axis_name="c", subcore_axis_name="s"),
    compiler_params=pltpu.CompilerParams(needs_layout_passes=False),
    scratch_types=[
        pltpu.VMEM((SEQ,), jnp.int32),
        pltpu.VMEM((CHUNK,), jnp.float32),
        pltpu.SemaphoreType.DMA,
    ],
)(_sc_pos_body)


def _split12(x):
    f = np.float32(x)
    bits = f.view(np.uint32) & np.uint32(0xFFFFF000)
    return bits.view(np.float32)


_PI_HI = _split12(np.pi)
_PI_MID = _split12(np.float64(np.pi) - np.float64(_PI_HI))
_INV_PI = np.float32(1.0 / np.pi)

_R = np.linspace(1e-7, np.pi / 2 + 0.01, 4001)
_U = _R * _R
_SIN_C = np.linalg.lstsq(
    np.stack([_U**j for j in range(3)], axis=1), np.sin(_R) / _R, rcond=None
)[0].astype(np.float32)
_COS_C = np.linalg.lstsq(
    np.stack([_U**j for j in range(4)], axis=1), np.cos(_R), rcond=None
)[0].astype(np.float32)


def _tc_body(pos_ref, freq_ref, out_ref):
    p_col = pos_ref[0]
    m_col = (p_col > 0.0).astype(jnp.float32)
    a = p_col * freq_ref[...]
    ki = (a * _INV_PI + jnp.float32(0.5)).astype(jnp.int32)
    k = ki.astype(jnp.float32)
    r = (a - k * _PI_HI) - k * _PI_MID
    u = r * r
    sinr = r * (_SIN_C[0] + u * (_SIN_C[1] + u * _SIN_C[2]))
    cosr = _COS_C[0] + u * (_COS_C[1] + u * (_COS_C[2] + u * _COS_C[3]))
    sgn = (jnp.float32(1.0) - jnp.float32(2.0) * (ki & 1).astype(jnp.float32)) * m_col
    out_ref[0] = jnp.concatenate([sinr * sgn, cosr * sgn], axis=1)


def _tc_stage(pos_f32):
    freqs = jnp.asarray(_FREQS).reshape(1, HALF_DIM)
    n_blocks = SEQ // SEQ_BLOCK
    return pl.pallas_call(
        _tc_body,
        grid=(BSZ, n_blocks),
        in_specs=[
            pl.BlockSpec((1, SEQ_BLOCK, 1), lambda b, s: (b, s, 0)),
            pl.BlockSpec((1, HALF_DIM), lambda b, s: (0, 0)),
        ],
        out_specs=pl.BlockSpec((1, SEQ_BLOCK, EMBEDDING_DIM), lambda b, s: (b, s, 0)),
        out_shape=jax.ShapeDtypeStruct((BSZ, SEQ, EMBEDDING_DIM), jnp.float32),
        compiler_params=pltpu.CompilerParams(
            dimension_semantics=("arbitrary", "arbitrary"),
        ),
    )(pos_f32, freqs)


@jax.jit
def kernel(pad_mask):
    bsz, seq_len = pad_mask.shape
    mask = jnp.logical_not(pad_mask).astype(jnp.int32).reshape(-1)
    pos = _sc_positions(mask)
    return _tc_stage(pos.reshape(bsz, seq_len, 1))


# speedup vs baseline: 1.0093x; 1.0093x over previous
import math
import functools

import jax
import jax.numpy as jnp
import numpy as np
from jax import lax
from jax.experimental import pallas as pl
from jax.experimental.pallas import tpu as pltpu
from jax.experimental.pallas import tpu_sc as plsc

BSZ = 4
SEQ = 8192
NUM_TOKENS = BSZ * SEQ
NW = 32
CHUNK = NUM_TOKENS // NW
CPR = SEQ // CHUNK
LANES = 16

EMBEDDING_DIM = 1024
HALF_DIM = EMBEDDING_DIM // 2
SEQ_BLOCK = 512

_EMB_SCALE = math.log(10000.0) / (HALF_DIM - 1)
_FREQS = np.exp(np.arange(HALF_DIM, dtype=np.float32) * -_EMB_SCALE).astype(np.float32)


def _sc_pos_body(mask_hbm, pos_hbm, row_v, pos_v, sem):
    wid = lax.axis_index("s") * 2 + lax.axis_index("c")
    b = wid // CPR
    c = wid % CPR
    row_base = b * SEQ
    cbase = c * CHUNK

    pltpu.sync_copy(mask_hbm.at[pl.ds(row_base, SEQ)], row_v)

    def _ofs(i, acc):
        return acc + jnp.sum(row_v[pl.ds(i * LANES, LANES)])

    carry = lax.fori_loop(0, c * (CHUNK // LANES), _ofs, jnp.int32(0))

    for i in range(CHUNK // LANES):
        v = row_v[pl.ds(cbase + i * LANES, LANES)]
        cum = plsc.cumsum(v) + carry
        carry = carry + jnp.sum(v)
        pos_v[pl.ds(i * LANES, LANES)] = (cum * v).astype(jnp.float32)

    pltpu.async_copy(pos_v, pos_hbm.at[pl.ds(row_base + cbase, CHUNK)], sem).wait()


_sc_positions = functools.partial(
    pl.kernel,
    out_type=jax.ShapeDtypeStruct((NUM_TOKENS,), jnp.float32),
    mesh=plsc.VectorSubcoreMesh(core_axis_name="c", subcore_axis_name="s"),
    compiler_params=pltpu.CompilerParams(needs_layout_passes=False),
    scratch_types=[
        pltpu.VMEM((SEQ,), jnp.int32),
        pltpu.VMEM((CHUNK,), jnp.float32),
        pltpu.SemaphoreType.DMA,
    ],
)(_sc_pos_body)


def _split12(x):
    f = np.float32(x)
    bits = f.view(np.uint32) & np.uint32(0xFFFFF000)
    return bits.view(np.float32)


_PI_HI = _split12(np.pi)
_PI_MID = _split12(np.float64(np.pi) - np.float64(_PI_HI))
_INV_PI = np.float32(1.0 / np.pi)

_R = np.linspace(1e-7, np.pi / 2 + 0.01, 4001)
_U = _R * _R
_SIN_C = np.linalg.lstsq(
    np.stack([_U**j for j in range(3)], axis=1), np.sin(_R) / _R, rcond=None
)[0].astype(np.float32)
_COS_C = np.linalg.lstsq(
    np.stack([_U**j for j in range(4)], axis=1), np.cos(_R), rcond=None
)[0].astype(np.float32)


_MAGIC = np.float32(12582912.0)


def _tc_body(pos_ref, freq_ref, out_ref):
    p_col = pos_ref[0]
    m_col = (p_col > 0.0).astype(jnp.float32)
    a = p_col * freq_ref[...]
    km = a * _INV_PI + _MAGIC
    k = km - _MAGIC
    par = jax.lax.bitcast_convert_type(km, jnp.int32) & 1
    r = (a - k * _PI_HI) - k * _PI_MID
    u = r * r
    sinr = r * (_SIN_C[0] + u * (_SIN_C[1] + u * _SIN_C[2]))
    cosr = _COS_C[0] + u * (_COS_C[1] + u * (_COS_C[2] + u * _COS_C[3]))
    sgn = (jnp.float32(1.0) - jnp.float32(2.0) * par.astype(jnp.float32)) * m_col
    out_ref[0, :, :HALF_DIM] = sinr * sgn
    out_ref[0, :, HALF_DIM:] = cosr * sgn


def _tc_stage(pos_f32):
    freqs = jnp.asarray(_FREQS).reshape(1, HALF_DIM)
    n_blocks = SEQ // SEQ_BLOCK
    return pl.pallas_call(
        _tc_body,
        grid=(BSZ, n_blocks),
        in_specs=[
            pl.BlockSpec((1, SEQ_BLOCK, 1), lambda b, s: (b, s, 0)),
            pl.BlockSpec((1, HALF_DIM), lambda b, s: (0, 0)),
        ],
        out_specs=pl.BlockSpec((1, SEQ_BLOCK, EMBEDDING_DIM), lambda b, s: (b, s, 0)),
        out_shape=jax.ShapeDtypeStruct((BSZ, SEQ, EMBEDDING_DIM), jnp.float32),
        compiler_params=pltpu.CompilerParams(
            dimension_semantics=("arbitrary", "arbitrary"),
        ),
    )(pos_f32, freqs)


@jax.jit
def kernel(pad_mask):
    bsz, seq_len = pad_mask.shape
    mask = jnp.logical_not(pad_mask).astype(jnp.int32).reshape(-1)
    pos = _sc_positions(mask)
    return _tc_stage(pos.reshape(bsz, seq_len, 1))
